# baseline (speedup 1.0000x reference)
import functools

import jax
import jax.numpy as jnp
import numpy as np
from jax.experimental import pallas as pl
from jax.experimental.pallas import tpu as pltpu

_NE = 64
_TK = 2
_CAPF = 1.25
_EPS = 1e-06
_TBLK = 2048
_NCHUNK = 8
_C = _TBLK // _NCHUNK
_NEG = -3.0e38


def _router_body(x_ref, w_ref, b_ref, lt_ref,
                 idx_ref, wout_ref, lb_ref, imp_ref, ovf_ref,
                 cnt_ref, imps_ref,
                 *, n_tokens, n_blocks, capacity):
    i = pl.program_id(0)

    @pl.when(i == 0)
    def _init():
        cnt_ref[...] = jnp.zeros_like(cnt_ref)
        imps_ref[...] = jnp.zeros_like(imps_ref)

    w = w_ref[...]
    bvec = b_ref[...]
    lt = lt_ref[...]
    cnt = cnt_ref[...]
    imp_acc = imps_ref[...]

    logits = []
    for c in range(_NCHUNK):
        xc = x_ref[pl.ds(c * _C, _C), :]
        lg = jax.lax.dot_general(w, xc, (((1,), (1,)), ((), ())),
                                 preferred_element_type=jnp.float32)
        logits.append(lg + bvec)

    for c in range(_NCHUNK):
        lg = logits[c]
        ioe = jax.lax.broadcasted_iota(jnp.int32, lg.shape, 0)
        m1 = jnp.max(lg, axis=0, keepdims=True)
        e0 = jnp.min(jnp.where(lg == m1, ioe, _NE), axis=0, keepdims=True)
        oh0 = ioe == e0
        l2 = jnp.where(oh0, _NEG, lg)
        m2 = jnp.max(l2, axis=0, keepdims=True)
        e1 = jnp.min(jnp.where(l2 == m2, ioe, _NE), axis=0, keepdims=True)
        oh1 = ioe == e1

        r = jnp.exp(m2 - m1)
        w0 = 1.0 / (1.0 + r)
        w1 = r * w0

        ex = jnp.exp(lg - m1)
        rs = 1.0 / jnp.sum(ex, axis=0, keepdims=True)
        imp_acc = imp_acc + ex * rs

        ohc = jnp.where(oh0 | oh1, 1.0, 0.0)
        pref = jax.lax.dot_general(ohc, lt, (((1,), (0,)), ((), ())),
                                   preferred_element_type=jnp.float32)
        km = (cnt + pref) < capacity
        keep0 = jnp.any(oh0 & km, axis=0, keepdims=True)
        keep1 = jnp.any(oh1 & km, axis=0, keepdims=True)

        cols = pl.ds(c * _C, _C)
        idx_ref[:, cols] = jnp.concatenate(
            [jnp.where(keep0, e0, -1), jnp.where(keep1, e1, -1)], axis=0)
        wout_ref[:, cols] = jnp.concatenate(
            [jnp.where(keep0, w0, 0.0), jnp.where(keep1, w1, 0.0)], axis=0)

        cnt = cnt + pref[:, _C - 1:_C] + ohc[:, _C - 1:_C]

    cnt_ref[...] = cnt
    imps_ref[...] = imp_acc

    @pl.when(i == n_blocks - 1)
    def _finish():
        nk = jnp.float32(n_tokens * _TK)
        ideal = jnp.float32(1.0 / _NE)
        kept = jnp.minimum(cnt, capacity)
        tpe = kept / nk
        lb_ref[...] = jnp.sum((tpe - ideal) ** 2, axis=0,
                              keepdims=True) / jnp.float32(_NE)
        impm = jnp.sum(imp_acc, axis=1, keepdims=True) / jnp.float32(n_tokens)
        imp_ref[...] = jnp.sum((impm - ideal) ** 2, axis=0,
                               keepdims=True) / jnp.float32(_NE)
        ovf_ref[...] = (nk - jnp.sum(kept, axis=0, keepdims=True)) / nk


def kernel(x, W, b):
    batch, seq, dim = x.shape
    xf = x.reshape(-1, dim)
    n = xf.shape[0]
    nb = n // _TBLK
    capacity = int(_CAPF * n * _TK / _NE)
    b2 = b.reshape(_NE, 1)
    lt = jnp.asarray(np.triu(np.ones((_C, _C), np.float32), k=1))

    out_shape = (
        jax.ShapeDtypeStruct((_TK, n), jnp.int32),
        jax.ShapeDtypeStruct((_TK, n), jnp.float32),
        jax.ShapeDtypeStruct((1, 1), jnp.float32),
        jax.ShapeDtypeStruct((1, 1), jnp.float32),
        jax.ShapeDtypeStruct((1, 1), jnp.float32),
    )
    in_specs = [
        pl.BlockSpec((_TBLK, dim), lambda i: (i, 0)),
        pl.BlockSpec((_NE, dim), lambda i: (0, 0)),
        pl.BlockSpec((_NE, 1), lambda i: (0, 0)),
        pl.BlockSpec((_C, _C), lambda i: (0, 0)),
    ]
    out_specs = (
        pl.BlockSpec((_TK, _TBLK), lambda i: (0, i)),
        pl.BlockSpec((_TK, _TBLK), lambda i: (0, i)),
        pl.BlockSpec((1, 1), lambda i: (0, 0)),
        pl.BlockSpec((1, 1), lambda i: (0, 0)),
        pl.BlockSpec((1, 1), lambda i: (0, 0)),
    )
    scratch_shapes = [
        pltpu.VMEM((_NE, 1), jnp.float32),
        pltpu.VMEM((_NE, _C), jnp.float32),
    ]
    idx2, w2, lb, imp, ovf = pl.pallas_call(
        functools.partial(_router_body, n_tokens=n, n_blocks=nb,
                          capacity=float(capacity)),
        grid=(nb,),
        in_specs=in_specs,
        out_specs=out_specs,
        out_shape=out_shape,
        scratch_shapes=scratch_shapes,
        compiler_params=pltpu.CompilerParams(
            dimension_semantics=("arbitrary",)),
    )(xf, W, b2, lt)
    return (idx2.T, w2.T, lb.reshape(()), imp.reshape(()), ovf.reshape(()))


# speedup vs baseline: 1.1199x; 1.1199x over previous
import functools

import jax
import jax.numpy as jnp
import numpy as np
from jax.experimental import pallas as pl
from jax.experimental.pallas import tpu as pltpu

_NE = 64
_TK = 2
_CAPF = 1.25
_EPS = 1e-06
_TBLK = 4096
_NCHUNK = 32
_C = _TBLK // _NCHUNK
_NEG = -3.0e38


def _router_body(x_ref, w_ref, b_ref, lt_ref,
                 idx_ref, wout_ref, lb_ref, imp_ref, ovf_ref,
                 cnt_ref, imps_ref,
                 *, n_tokens, n_blocks, capacity):
    i = pl.program_id(0)

    @pl.when(i == 0)
    def _init():
        cnt_ref[...] = jnp.zeros_like(cnt_ref)
        imps_ref[...] = jnp.zeros_like(imps_ref)

    w = w_ref[...]
    bvec = b_ref[...]
    lt = lt_ref[...]
    cnt = cnt_ref[...]
    imp_acc = imps_ref[...]

    logits = []
    for c in range(_NCHUNK):
        xc = x_ref[pl.ds(c * _C, _C), :]
        lg = jax.lax.dot_general(w, xc, (((1,), (1,)), ((), ())),
                                 preferred_element_type=jnp.float32)
        logits.append(lg + bvec)

    for c in range(_NCHUNK):
        lg = logits[c]
        ioe = jax.lax.broadcasted_iota(jnp.int32, lg.shape, 0)
        m1 = jnp.max(lg, axis=0, keepdims=True)
        e0 = jnp.min(jnp.where(lg == m1, ioe, _NE), axis=0, keepdims=True)
        oh0 = ioe == e0
        l2 = jnp.where(oh0, _NEG, lg)
        m2 = jnp.max(l2, axis=0, keepdims=True)
        e1 = jnp.min(jnp.where(l2 == m2, ioe, _NE), axis=0, keepdims=True)
        oh1 = ioe == e1

        r = jnp.exp(m2 - m1)
        w0 = 1.0 / (1.0 + r)
        w1 = r * w0

        ex = jnp.exp(lg - m1)
        rs = 1.0 / jnp.sum(ex, axis=0, keepdims=True)
        imp_acc = imp_acc + ex * rs

        ohc = jnp.where(oh0 | oh1, 1.0, 0.0)
        pref = jax.lax.dot_general(ohc, lt, (((1,), (0,)), ((), ())),
                                   preferred_element_type=jnp.float32)
        km = (cnt + pref) < capacity
        keep0 = jnp.any(oh0 & km, axis=0, keepdims=True)
        keep1 = jnp.any(oh1 & km, axis=0, keepdims=True)

        cols = pl.ds(c * _C, _C)
        idx_ref[:, cols] = jnp.concatenate(
            [jnp.where(keep0, e0, -1), jnp.where(keep1, e1, -1)], axis=0)
        wout_ref[:, cols] = jnp.concatenate(
            [jnp.where(keep0, w0, 0.0), jnp.where(keep1, w1, 0.0)], axis=0)

        cnt = cnt + pref[:, _C - 1:_C] + ohc[:, _C - 1:_C]

    cnt_ref[...] = cnt
    imps_ref[...] = imp_acc

    @pl.when(i == n_blocks - 1)
    def _finish():
        nk = jnp.float32(n_tokens * _TK)
        ideal = jnp.float32(1.0 / _NE)
        kept = jnp.minimum(cnt, capacity)
        tpe = kept / nk
        lb_ref[...] = jnp.sum((tpe - ideal) ** 2, axis=0,
                              keepdims=True) / jnp.float32(_NE)
        impm = jnp.sum(imp_acc, axis=1, keepdims=True) / jnp.float32(n_tokens)
        imp_ref[...] = jnp.sum((impm - ideal) ** 2, axis=0,
                               keepdims=True) / jnp.float32(_NE)
        ovf_ref[...] = (nk - jnp.sum(kept, axis=0, keepdims=True)) / nk


def kernel(x, W, b):
    batch, seq, dim = x.shape
    xf = x.reshape(-1, dim)
    n = xf.shape[0]
    nb = n // _TBLK
    capacity = int(_CAPF * n * _TK / _NE)
    b2 = b.reshape(_NE, 1)
    lt = jnp.asarray(np.triu(np.ones((_C, _C), np.float32), k=1))

    out_shape = (
        jax.ShapeDtypeStruct((_TK, n), jnp.int32),
        jax.ShapeDtypeStruct((_TK, n), jnp.float32),
        jax.ShapeDtypeStruct((1, 1), jnp.float32),
        jax.ShapeDtypeStruct((1, 1), jnp.float32),
        jax.ShapeDtypeStruct((1, 1), jnp.float32),
    )
    in_specs = [
        pl.BlockSpec((_TBLK, dim), lambda i: (i, 0)),
        pl.BlockSpec((_NE, dim), lambda i: (0, 0)),
        pl.BlockSpec((_NE, 1), lambda i: (0, 0)),
        pl.BlockSpec((_C, _C), lambda i: (0, 0)),
    ]
    out_specs = (
        pl.BlockSpec((_TK, _TBLK), lambda i: (0, i)),
        pl.BlockSpec((_TK, _TBLK), lambda i: (0, i)),
        pl.BlockSpec((1, 1), lambda i: (0, 0)),
        pl.BlockSpec((1, 1), lambda i: (0, 0)),
        pl.BlockSpec((1, 1), lambda i: (0, 0)),
    )
    scratch_shapes = [
        pltpu.VMEM((_NE, 1), jnp.float32),
        pltpu.VMEM((_NE, _C), jnp.float32),
    ]
    idx2, w2, lb, imp, ovf = pl.pallas_call(
        functools.partial(_router_body, n_tokens=n, n_blocks=nb,
                          capacity=float(capacity)),
        grid=(nb,),
        in_specs=in_specs,
        out_specs=out_specs,
        out_shape=out_shape,
        scratch_shapes=scratch_shapes,
        compiler_params=pltpu.CompilerParams(
            dimension_semantics=("arbitrary",)),
    )(xf, W, b2, lt)
    return (idx2.T, w2.T, lb.reshape(()), imp.reshape(()), ovf.reshape(()))
